# Initial kernel scaffold; baseline (speedup 1.0000x reference)
import jax
import jax.numpy as jnp
from jax.experimental import pallas as pl


def kernel(x, edge_index, pos_emb, lap_pe, W_pos, b_pos, W_xemb, b_xemb):
    raise NotImplementedError("write your pallas kernel here")


# speedup vs baseline: 10.3674x; 10.3674x over previous
import jax
import jax.numpy as jnp
from jax import lax
from jax.experimental import pallas as pl
from jax.experimental.pallas import tpu as pltpu
from jax.experimental.pallas import tpu_sc as plsc

N_NODES = 10000
N_EDGES = 320000
D_FEAT = 128
HIDDEN = 128

NC = 2
NS = 16
NW = NC * NS
E_PER_TILE = N_EDGES // NW
CHUNK = 80
NCHUNK = E_PER_TILE // CHUNK
NP = 10240
ROWS_PER_TILE = NP // NS
DEG_W = 16

_mesh = lambda: plsc.VectorSubcoreMesh(core_axis_name="c", subcore_axis_name="s")
_sc_params = lambda: pltpu.CompilerParams(use_tc_tiling_on_sc=False)


def _deg_body(dst_hbm, zeros_hbm, ones_hbm, out_hbm, didx, ones_v, accum):
    c = lax.axis_index("c")
    s = lax.axis_index("s")
    wid = s * NC + c
    pltpu.sync_copy(dst_hbm.at[wid], didx)
    pltpu.sync_copy(ones_hbm, ones_v)
    r0 = s * ROWS_PER_TILE
    pltpu.sync_copy(zeros_hbm.at[pl.ds(r0, ROWS_PER_TILE)],
                    accum.at[pl.ds(r0, ROWS_PER_TILE)])
    plsc.subcore_barrier()

    @pl.loop(0, NCHUNK)
    def _(j):
        pltpu.sync_copy(ones_v, accum.at[didx.at[j]], add=True)

    plsc.subcore_barrier()
    pltpu.sync_copy(accum.at[pl.ds(r0, ROWS_PER_TILE)],
                    out_hbm.at[c, pl.ds(r0, ROWS_PER_TILE)])


def _deg_call(dst_r, zeros16, ones16):
    f = pl.kernel(
        _deg_body,
        out_type=jax.ShapeDtypeStruct((NC, NP, DEG_W), jnp.float32),
        mesh=_mesh(),
        compiler_params=_sc_params(),
        scratch_types=[
            pltpu.VMEM((NCHUNK, CHUNK), jnp.int32),
            pltpu.VMEM((CHUNK, DEG_W), jnp.float32),
            pltpu.VMEM_SHARED((NP, DEG_W), jnp.float32),
        ],
    )
    return f(dst_r, zeros16, ones16)


def _spmm_body(g_hbm, src_hbm, dst_hbm, out_hbm,
               sidx, didx, buf0, buf1, accum, sem0, sem1):
    c = lax.axis_index("c")
    s = lax.axis_index("s")
    wid = s * NC + c
    pltpu.sync_copy(src_hbm.at[wid], sidx)
    pltpu.sync_copy(dst_hbm.at[wid], didx)
    r0 = s * ROWS_PER_TILE
    pltpu.sync_copy(g_hbm.at[pl.ds(r0, ROWS_PER_TILE)],
                    accum.at[pl.ds(r0, ROWS_PER_TILE)])
    plsc.subcore_barrier()

    pltpu.async_copy(g_hbm.at[sidx.at[0]], buf0, sem0)

    @pl.loop(0, NCHUNK - 1, step=2)
    def _(j):
        pltpu.async_copy(g_hbm.at[sidx.at[j + 1]], buf1, sem1)
        pltpu.make_async_copy(g_hbm.at[sidx.at[j]], buf0, sem0).wait()
        pltpu.sync_copy(buf0, accum.at[didx.at[j]], add=True)
        pltpu.async_copy(g_hbm.at[sidx.at[j + 2]], buf0, sem0)
        pltpu.make_async_copy(g_hbm.at[sidx.at[j + 1]], buf1, sem1).wait()
        pltpu.sync_copy(buf1, accum.at[didx.at[j + 1]], add=True)

    pltpu.make_async_copy(g_hbm.at[sidx.at[NCHUNK - 1]], buf0, sem0).wait()
    pltpu.sync_copy(buf0, accum.at[didx.at[NCHUNK - 1]], add=True)

    plsc.subcore_barrier()
    pltpu.sync_copy(accum.at[pl.ds(r0, ROWS_PER_TILE)],
                    out_hbm.at[c, pl.ds(r0, ROWS_PER_TILE)])


def _spmm_call(g, src_r, dst_r):
    f = pl.kernel(
        _spmm_body,
        out_type=jax.ShapeDtypeStruct((NC, NP, HIDDEN), jnp.float32),
        mesh=_mesh(),
        compiler_params=_sc_params(),
        scratch_types=[
            pltpu.VMEM((NCHUNK, CHUNK), jnp.int32),
            pltpu.VMEM((NCHUNK, CHUNK), jnp.int32),
            pltpu.VMEM((CHUNK, HIDDEN), jnp.float32),
            pltpu.VMEM((CHUNK, HIDDEN), jnp.float32),
            pltpu.VMEM_SHARED((NP, HIDDEN), jnp.float32),
            pltpu.SemaphoreType.DMA,
            pltpu.SemaphoreType.DMA,
        ],
    )
    return f(g, src_r, dst_r)


def _prologue_body(x_ref, pos_ref, lap_ref, wpa_ref, wpb_ref, bp_ref,
                   wxt_ref, wxb_ref, bx_ref, degp_ref,
                   g0_ref, norm_ref, norm2_ref):
    pe = (jnp.dot(pos_ref[...], wpa_ref[...], preferred_element_type=jnp.float32)
          + jnp.dot(lap_ref[...], wpb_ref[...], preferred_element_type=jnp.float32)
          + bp_ref[...])
    h = (jnp.dot(x_ref[...], wxt_ref[...], preferred_element_type=jnp.float32)
         + jnp.dot(pe, wxb_ref[...], preferred_element_type=jnp.float32)
         + bx_ref[...])
    deg = (degp_ref[0] + degp_ref[1])[:N_NODES, 0:1]
    norm = lax.rsqrt(1.0 + deg)
    norm_ref[:N_NODES] = norm
    norm_ref[N_NODES:] = jnp.zeros((NP - N_NODES, 1), jnp.float32)
    norm2_ref[:N_NODES] = norm * norm
    norm2_ref[N_NODES:] = jnp.zeros((NP - N_NODES, 1), jnp.float32)
    g0_ref[:N_NODES] = h * norm
    g0_ref[N_NODES:] = jnp.zeros((NP - N_NODES, HIDDEN), jnp.float32)


def _prologue_call(x, pos, lap, wpa, wpb, bp, wxt, wxb, bx, degp):
    return pl.pallas_call(
        _prologue_body,
        out_shape=[
            jax.ShapeDtypeStruct((NP, HIDDEN), jnp.float32),
            jax.ShapeDtypeStruct((NP, 1), jnp.float32),
            jax.ShapeDtypeStruct((NP, 1), jnp.float32),
        ],
    )(x, pos, lap, wpa, wpb, bp, wxt, wxb, bx, degp)


def _combine_body(p_ref, g_ref, s_ref, out_ref):
    out_ref[...] = s_ref[...] * (p_ref[0] + p_ref[1] - g_ref[...])


def _combine_call(p, g, scale):
    return pl.pallas_call(
        _combine_body,
        out_shape=jax.ShapeDtypeStruct((NP, HIDDEN), jnp.float32),
    )(p, g, scale)


def kernel(x, edge_index, pos_emb, lap_pe, W_pos, b_pos, W_xemb, b_xemb):
    dst = edge_index[0].astype(jnp.int32).reshape(NW, NCHUNK, CHUNK)
    src = edge_index[1].astype(jnp.int32).reshape(NW, NCHUNK, CHUNK)

    wpa = W_pos[:pos_emb.shape[1]]
    wpb = W_pos[pos_emb.shape[1]:]
    wxt = W_xemb[:D_FEAT]
    wxb = W_xemb[D_FEAT:]

    zeros16 = jnp.zeros((NP, DEG_W), jnp.float32)
    ones16 = jnp.ones((CHUNK, DEG_W), jnp.float32)

    degp = _deg_call(dst, zeros16, ones16)
    g, norm, norm2 = _prologue_call(x, pos_emb, lap_pe, wpa, wpb, b_pos,
                                    wxt, wxb, b_xemb, degp)
    for layer in range(3):
        p = _spmm_call(g, src, dst)
        g = _combine_call(p, g, norm if layer == 2 else norm2)
    return g[:N_NODES]
